# baseline (speedup 1.0000x reference)
import jax
import jax.numpy as jnp
from jax.experimental import pallas as pl

_HIDDEN = 2048
_OUT = 3072
_Q = 2048
_KV = 512
_R = 16
_SCALING = 2.0
_TM = 512


def _fused_kernel(x_ref, w_ref, a_ref, b_ref, o_ref):
    xt = x_ref[...].astype(jnp.bfloat16)
    base = jnp.dot(xt, w_ref[...], preferred_element_type=jnp.float32)
    xa = jnp.dot(xt, a_ref[...], preferred_element_type=jnp.float32)
    delta = jnp.dot(xa.astype(jnp.bfloat16), b_ref[...],
                    preferred_element_type=jnp.float32)
    o_ref[...] = base + delta * _SCALING


def kernel(x, weight, lora_A, lora_B_q, lora_B_k, lora_B_v):
    orig_shape = x.shape
    x_flat = x.reshape(-1, _HIDDEN)
    n = x_flat.shape[0]

    wt = weight.T.astype(jnp.bfloat16)
    a48t = lora_A[0].reshape(3 * _R, _HIDDEN).T.astype(jnp.bfloat16)
    bbdt = jnp.zeros((3 * _R, _OUT), dtype=jnp.bfloat16)
    bbdt = bbdt.at[:_R, :_Q].set(lora_B_q[0].T.astype(jnp.bfloat16))
    bbdt = bbdt.at[_R:2 * _R, _Q:_Q + _KV].set(lora_B_k[0].T.astype(jnp.bfloat16))
    bbdt = bbdt.at[2 * _R:, _Q + _KV:].set(lora_B_v[0].T.astype(jnp.bfloat16))

    grid = (n // _TM,)
    out = pl.pallas_call(
        _fused_kernel,
        grid=grid,
        in_specs=[
            pl.BlockSpec((_TM, _HIDDEN), lambda i: (i, 0)),
            pl.BlockSpec((_HIDDEN, _OUT), lambda i: (0, 0)),
            pl.BlockSpec((_HIDDEN, 3 * _R), lambda i: (0, 0)),
            pl.BlockSpec((3 * _R, _OUT), lambda i: (0, 0)),
        ],
        out_specs=pl.BlockSpec((_TM, _OUT), lambda i: (i, 0)),
        out_shape=jax.ShapeDtypeStruct((n, _OUT), jnp.float32),
    )(x_flat, wt, a48t, bbdt)
    return out.reshape(*orig_shape[:-1], _OUT)


# speedup vs baseline: 1.2165x; 1.2165x over previous
import jax
import jax.numpy as jnp
from jax.experimental import pallas as pl

_HIDDEN = 2048
_OUT = 3072
_Q = 2048
_KV = 512
_R = 16
_SCALING = 2.0
_TM = 512


def _fold_kernel(w_ref, a_ref, b_ref, weff_ref):
    delta = jax.lax.dot_general(
        b_ref[...], a_ref[...], (((1,), (0,)), ((), ())),
        preferred_element_type=jnp.float32)
    weff_ref[...] = (w_ref[...] + delta * _SCALING).astype(jnp.bfloat16)


def _matmul_kernel(x_ref, w_ref, o_ref):
    xt = x_ref[...].astype(jnp.bfloat16)
    o_ref[...] = jax.lax.dot_general(
        xt, w_ref[...], (((1,), (1,)), ((), ())),
        preferred_element_type=jnp.float32)


def kernel(x, weight, lora_A, lora_B_q, lora_B_k, lora_B_v):
    orig_shape = x.shape
    x_flat = x.reshape(-1, _HIDDEN)
    n = x_flat.shape[0]

    a48 = lora_A[0].reshape(3 * _R, _HIDDEN)
    bbd = jnp.zeros((_OUT, 3 * _R), dtype=jnp.float32)
    bbd = bbd.at[:_Q, :_R].set(lora_B_q[0])
    bbd = bbd.at[_Q:_Q + _KV, _R:2 * _R].set(lora_B_k[0])
    bbd = bbd.at[_Q + _KV:, 2 * _R:].set(lora_B_v[0])

    weff = pl.pallas_call(
        _fold_kernel,
        out_shape=jax.ShapeDtypeStruct((_OUT, _HIDDEN), jnp.bfloat16),
    )(weight, a48, bbd)

    out = pl.pallas_call(
        _matmul_kernel,
        grid=(n // _TM,),
        in_specs=[
            pl.BlockSpec((_TM, _HIDDEN), lambda i: (i, 0)),
            pl.BlockSpec((_OUT, _HIDDEN), lambda i: (0, 0)),
        ],
        out_specs=pl.BlockSpec((_TM, _OUT), lambda i: (i, 0)),
        out_shape=jax.ShapeDtypeStruct((n, _OUT), jnp.float32),
    )(x_flat, weff)
    return out.reshape(*orig_shape[:-1], _OUT)


# speedup vs baseline: 1.2214x; 1.0040x over previous
import jax
import jax.numpy as jnp
from jax.experimental import pallas as pl

_HIDDEN = 2048
_OUT = 3072
_Q = 2048
_KV = 512
_R = 16
_SCALING = 2.0
_TM = 1024


def _fold_kernel(w_ref, a_ref, b_ref, weff_ref):
    delta = jax.lax.dot_general(
        b_ref[...], a_ref[...], (((1,), (0,)), ((), ())),
        preferred_element_type=jnp.float32)
    weff_ref[...] = (w_ref[...] + delta * _SCALING).astype(jnp.bfloat16)


def _matmul_kernel(x_ref, w_ref, o_ref):
    xt = x_ref[...].astype(jnp.bfloat16)
    o_ref[...] = jax.lax.dot_general(
        xt, w_ref[...], (((1,), (1,)), ((), ())),
        preferred_element_type=jnp.float32)


def kernel(x, weight, lora_A, lora_B_q, lora_B_k, lora_B_v):
    orig_shape = x.shape
    x_flat = x.reshape(-1, _HIDDEN)
    n = x_flat.shape[0]

    a48 = lora_A[0].reshape(3 * _R, _HIDDEN)
    bbd = jnp.zeros((_OUT, 3 * _R), dtype=jnp.float32)
    bbd = bbd.at[:_Q, :_R].set(lora_B_q[0])
    bbd = bbd.at[_Q:_Q + _KV, _R:2 * _R].set(lora_B_k[0])
    bbd = bbd.at[_Q + _KV:, 2 * _R:].set(lora_B_v[0])

    weff = pl.pallas_call(
        _fold_kernel,
        out_shape=jax.ShapeDtypeStruct((_OUT, _HIDDEN), jnp.bfloat16),
    )(weight, a48, bbd)

    out = pl.pallas_call(
        _matmul_kernel,
        grid=(n // _TM,),
        in_specs=[
            pl.BlockSpec((_TM, _HIDDEN), lambda i: (i, 0)),
            pl.BlockSpec((_OUT, _HIDDEN), lambda i: (0, 0)),
        ],
        out_specs=pl.BlockSpec((_TM, _OUT), lambda i: (i, 0)),
        out_shape=jax.ShapeDtypeStruct((n, _OUT), jnp.float32),
    )(x_flat, weff)
    return out.reshape(*orig_shape[:-1], _OUT)
